# baseline (speedup 1.0000x reference)
import functools

import jax
import jax.numpy as jnp
from jax import lax
from jax.experimental import pallas as pl
from jax.experimental.pallas import tpu as pltpu
from jax.experimental.pallas import tpu_sc as plsc

_DIM = 64
_W = 200
_RPC = 2
_C = _RPC * _W
_SUPC = 4
_SUPI = _SUPC * _C


@functools.partial(jax.jit, static_argnames=("rows", "width"))
def _lookup(x_flat, table, rows, width):
    info = plsc.get_sparse_core_info()
    nw = info.num_cores * info.num_subcores
    b_per_w = rows * width // nw
    rows_per_w = rows // nw
    n_sup = b_per_w // _SUPI
    mesh = plsc.VectorSubcoreMesh(core_axis_name="c", subcore_axis_name="s")

    @functools.partial(
        pl.kernel,
        mesh=mesh,
        compiler_params=pltpu.CompilerParams(needs_layout_passes=False),
        out_type=jax.ShapeDtypeStruct((rows, width, _DIM), jnp.float32),
        scratch_types=[
            pltpu.VMEM((128 * _DIM,), jnp.float32),
            pltpu.VMEM((_SUPI,), jnp.int32),
            pltpu.VMEM((2 * _C, _DIM), jnp.float32),
            pltpu.SemaphoreType.DMA((2,)),
        ],
    )
    def k(x_hbm, table_hbm, out_hbm, tab_flat, idx_v, rows_v, wsem):
        wid = lax.axis_index("s") * info.num_cores + lax.axis_index("c")
        base = wid * b_per_w
        row_base = wid * rows_per_w
        lanes = lax.iota(jnp.int32, 16)

        pltpu.sync_copy(table_hbm, rows_v.at[pl.ds(0, 128)])

        def flat_body(v, _):
            for c in range(_DIM // 16):
                tab_flat[pl.ds(v * _DIM + c * 16, 16)] = rows_v[
                    v, pl.ds(c * 16, 16)
                ]
            return 0

        lax.fori_loop(0, 128, flat_body, 0)

        def sup_body(s, _):
            off = base + s * _SUPI
            pltpu.sync_copy(x_hbm.at[pl.ds(off, _SUPI)], idx_v)

            def chunk_body(g, _):
                i_glob = s * _SUPC + g
                buf = lax.rem(i_glob, 2)

                @pl.when(i_glob >= 2)
                def _wait_prev():
                    for kk in range(_RPC):
                        pltpu.make_async_copy(
                            rows_v.at[pl.ds(buf * _C + kk * _W, _W)],
                            out_hbm.at[0],
                            wsem.at[buf],
                        ).wait()

                @plsc.parallel_loop(0, _C // 16)
                def grp_body(q):
                    ivec = idx_v[pl.ds(g * _C + q * 16, 16)]
                    avec = ivec * _DIM
                    rvec = buf * _C + q * 16 + lanes
                    for c0 in range(0, _DIM, 16):
                        cols = [
                            (lanes + c) & (_DIM - 1)
                            for c in range(c0, c0 + 16)
                        ]
                        vals = [
                            plsc.load_gather(tab_flat, [avec + colv])
                            for colv in cols
                        ]
                        for colv, v in zip(cols, vals):
                            plsc.store_scatter(rows_v, [rvec, colv], v)

                xrow = row_base + i_glob * _RPC
                for kk in range(_RPC):
                    pltpu.async_copy(
                        rows_v.at[pl.ds(buf * _C + kk * _W, _W)],
                        out_hbm.at[xrow + kk],
                        wsem.at[buf],
                    )
                return 0

            lax.fori_loop(0, _SUPC, chunk_body, 0)
            return 0

        lax.fori_loop(0, n_sup, sup_body, 0)

        for b in range(2):
            for kk in range(_RPC):
                pltpu.make_async_copy(
                    rows_v.at[pl.ds(b * _C + kk * _W, _W)],
                    out_hbm.at[0],
                    wsem.at[b],
                ).wait()

    return k(x_flat, table)


def kernel(x, table):
    x_flat = jnp.ravel(x).astype(jnp.int32)
    return _lookup(x_flat, table, x.shape[0], x.shape[1])


# speedup vs baseline: 4.9867x; 4.9867x over previous
import functools

import jax
import jax.numpy as jnp
from jax import lax
from jax.experimental import pallas as pl
from jax.experimental.pallas import tpu as pltpu
from jax.experimental.pallas import tpu_sc as plsc

_DIM = 64
_IC = 4096


@functools.partial(jax.jit, static_argnames=("rows", "width"))
def _lookup(x_t, table_t, rows, width):
    info = plsc.get_sparse_core_info()
    nw = info.num_cores * info.num_subcores
    n_strips = width * (_DIM // 8)
    strips_per_w = n_strips // nw
    n_ic = rows // _IC
    mesh = plsc.VectorSubcoreMesh(core_axis_name="c", subcore_axis_name="s")

    @functools.partial(
        pl.kernel,
        mesh=mesh,
        compiler_params=pltpu.CompilerParams(needs_layout_passes=False),
        out_type=jax.ShapeDtypeStruct((width, _DIM, rows), jnp.float32),
        scratch_types=[
            pltpu.VMEM((_DIM, 128), jnp.float32),
            pltpu.VMEM((rows,), jnp.int32),
            pltpu.VMEM((2, 8, _IC), jnp.float32),
            pltpu.SemaphoreType.DMA((2,)),
        ],
    )
    def k(xt_hbm, tabt_hbm, out_hbm, tabt_v, idx_col, strip_v, wsem):
        wid = lax.axis_index("s") * info.num_cores + lax.axis_index("c")
        s0 = wid * strips_per_w
        pltpu.sync_copy(tabt_hbm, tabt_v)

        def strip_body(s, _):
            sid = s0 + s
            j = sid // 8
            dt = lax.rem(sid, 8)

            @pl.when(jnp.logical_or(s == 0, dt == 0))
            def _stage_col():
                pltpu.sync_copy(xt_hbm.at[j], idx_col)

            def ic_body(ic, _):
                cidx = s * n_ic + ic
                buf = lax.rem(cidx, 2)

                @pl.when(cidx >= 2)
                def _wait_prev():
                    pltpu.make_async_copy(
                        strip_v.at[buf],
                        out_hbm.at[0, pl.ds(0, 8), pl.ds(0, _IC)],
                        wsem.at[buf],
                    ).wait()

                @plsc.parallel_loop(0, _IC // 16)
                def g_body(g):
                    ivec = idx_col[pl.ds(ic * _IC + g * 16, 16)]
                    for dd in range(8):
                        dsplat = jnp.full((16,), dt * 8 + dd, jnp.int32)
                        vals = plsc.load_gather(tabt_v, [dsplat, ivec])
                        strip_v[buf, dd, pl.ds(g * 16, 16)] = vals

                pltpu.async_copy(
                    strip_v.at[buf],
                    out_hbm.at[j, pl.ds(dt * 8, 8), pl.ds(ic * _IC, _IC)],
                    wsem.at[buf],
                )
                return 0

            lax.fori_loop(0, n_ic, ic_body, 0)
            return 0

        lax.fori_loop(0, strips_per_w, strip_body, 0)

        for b in range(2):
            pltpu.make_async_copy(
                strip_v.at[b],
                out_hbm.at[0, pl.ds(0, 8), pl.ds(0, _IC)],
                wsem.at[b],
            ).wait()

    return k(x_t, table_t)


def kernel(x, table):
    x_t = jnp.swapaxes(x, 0, 1).astype(jnp.int32)
    table_t = jnp.swapaxes(table, 0, 1)
    out_t = _lookup(x_t, table_t, x.shape[0], x.shape[1])
    return jnp.transpose(out_t, (2, 0, 1))
